# Initial kernel scaffold; baseline (speedup 1.0000x reference)
import jax
import jax.numpy as jnp
from jax.experimental import pallas as pl


def kernel(x, weights):
    raise NotImplementedError("write your pallas kernel here")


# speedup vs baseline: 8.1311x; 8.1311x over previous
import functools

import jax
import jax.numpy as jnp
from jax import lax
from jax.experimental import pallas as pl
from jax.experimental.pallas import tpu as pltpu
from jax.experimental.pallas import tpu_sc as plsc

_NC = 2
_NS = 16
_NW = _NC * _NS
_CHUNK = 512
_LANES = 128


def _gather_body(idx_hbm, table_hbm, out_hbm, shared_table, idx_v, rows_v, sem):
    cid = lax.axis_index("c")
    sid = lax.axis_index("s")
    wid = sid * _NC + cid
    b_total = out_hbm.shape[0]
    emb = out_hbm.shape[1]
    b_per_w = b_total // _NW
    steps = b_per_w // _CHUNK
    base = wid * b_per_w

    @pl.when(sid == 0)
    def _stage():
        pltpu.sync_copy(table_hbm, shared_table)

    plsc.subcore_barrier()

    def body(j, carry):
        off = base + j * _CHUNK
        pltpu.sync_copy(idx_hbm.at[pl.ds(off, _CHUNK)], idx_v)
        pltpu.async_copy(shared_table.at[idx_v], rows_v, sem).wait()
        pltpu.sync_copy(rows_v, out_hbm.at[pl.ds(off, _CHUNK)])
        return carry

    lax.fori_loop(0, steps, body, 0)


def kernel(x, weights):
    b, h = x.shape
    v, emb = weights.shape
    flat_idx = x.reshape(b * h)
    run = functools.partial(
        pl.kernel,
        mesh=plsc.VectorSubcoreMesh(core_axis_name="c", subcore_axis_name="s"),
        out_type=jax.ShapeDtypeStruct((b * h, emb), jnp.float32),
        scratch_types=[
            pltpu.VMEM_SHARED((v, emb), jnp.float32),
            pltpu.VMEM((_CHUNK,), jnp.int32),
            pltpu.VMEM((_CHUNK, emb), jnp.float32),
            pltpu.SemaphoreType.DMA,
        ],
    )(_gather_body)
    out = run(flat_idx, weights)
    return out.reshape(b, h, emb)
